# Initial kernel scaffold; baseline (speedup 1.0000x reference)
import jax
import jax.numpy as jnp
from jax.experimental import pallas as pl


def kernel(x, edge_index, W1, b1, W2, b2, W3, b3, W4, b4, Wf, bf, scale_weights):
    raise NotImplementedError("write your pallas kernel here")


# speedup vs baseline: 7.8429x; 7.8429x over previous
import functools

import jax
import jax.numpy as jnp
from jax import lax
from jax.experimental import pallas as pl
from jax.experimental.pallas import tpu as pltpu
from jax.experimental.pallas import tpu_sc as plsc

N = 10000
E = 320000
IN = 128
HID = 256
OUT = 128

NC = 2
NS = 16
CH = 128

ROWS = 10240
RPS = ROWS // NS
DUMMY = N

EPS = -(-(E // NS) // CH) * CH
EPW = -(-(E // (NC * NS)) // CH) * CH

RB = 400
GRID = N // RB


@functools.lru_cache(maxsize=None)
def _make_sc_agg(width):
  nchunk = EPS // CH

  @functools.partial(
      pl.kernel,
      out_type=jax.ShapeDtypeStruct((NC, ROWS, width), jnp.float32),
      mesh=plsc.VectorSubcoreMesh(core_axis_name="c", subcore_axis_name="s",
                                  num_cores=NC, num_subcores=NS),
      scratch_types=[
          pltpu.VMEM((CH,), jnp.int32),
          pltpu.VMEM((CH,), jnp.int32),
          pltpu.VMEM((CH, width), jnp.float32),
          pltpu.VMEM_SHARED((ROWS, width), jnp.float32),
          pltpu.SemaphoreType.DMA,
      ],
  )
  def agg(table, srcs, dsts, out, sidx, didx, buf, acc, sem):
    c = lax.axis_index("c")
    s = lax.axis_index("s")

    z16 = jnp.zeros((16,), jnp.float32)

    def zrow(i, _):
      for j in range(width // 16):
        buf[i, pl.ds(j * 16, 16)] = z16
      return 0

    lax.fori_loop(0, CH, zrow, 0)
    for k in range(RPS // CH):
      pltpu.sync_copy(buf, acc.at[pl.ds(s * RPS + k * CH, CH)])
    plsc.subcore_barrier()

    ebase = s * EPS

    def body(g, _):
      base = ebase + g * CH
      pltpu.sync_copy(srcs.at[c, pl.ds(base, CH)], sidx)
      pltpu.sync_copy(dsts.at[pl.ds(base, CH)], didx)
      pltpu.async_copy(table.at[sidx], buf, sem).wait()
      pltpu.sync_copy(buf, acc.at[didx], add=True)
      return 0

    lax.fori_loop(0, nchunk, body, 0)
    plsc.subcore_barrier()

    for k in range(RPS // CH):
      r = s * RPS + k * CH
      pltpu.sync_copy(acc.at[pl.ds(r, CH)], buf)
      pltpu.sync_copy(buf, out.at[c, pl.ds(r, CH)])

  return agg


@functools.lru_cache(maxsize=None)
def _make_sc_agg_full():
  W = OUT
  nchunk = EPW // CH

  @functools.partial(
      pl.kernel,
      out_type=jax.ShapeDtypeStruct((NC, ROWS, W), jnp.float32),
      mesh=plsc.VectorSubcoreMesh(core_axis_name="c", subcore_axis_name="s",
                                  num_cores=NC, num_subcores=NS),
      scratch_types=[
          pltpu.VMEM((CH,), jnp.int32),
          pltpu.VMEM((CH,), jnp.int32),
          pltpu.VMEM((CH, W), jnp.float32),
          pltpu.VMEM_SHARED((ROWS, W), jnp.float32),
          pltpu.SemaphoreType.DMA,
      ],
  )
  def agg(table, srcs, dsts, out, sidx, didx, buf, acc, sem):
    c = lax.axis_index("c")
    s = lax.axis_index("s")
    z16 = jnp.zeros((16,), jnp.float32)

    def zrow(i, _):
      for j in range(W // 16):
        buf[i, pl.ds(j * 16, 16)] = z16
      return 0

    lax.fori_loop(0, CH, zrow, 0)
    for k in range(RPS // CH):
      pltpu.sync_copy(buf, acc.at[pl.ds(s * RPS + k * CH, CH)])
    plsc.subcore_barrier()

    ebase = s * EPW

    def body(g, _):
      base = ebase + g * CH
      pltpu.sync_copy(srcs.at[c, pl.ds(base, CH)], sidx)
      pltpu.sync_copy(dsts.at[c, pl.ds(base, CH)], didx)
      pltpu.async_copy(table.at[sidx], buf, sem).wait()
      pltpu.sync_copy(buf, acc.at[didx], add=True)
      return 0

    lax.fori_loop(0, nchunk, body, 0)
    plsc.subcore_barrier()

    for k in range(RPS // CH):
      r = s * RPS + k * CH
      pltpu.sync_copy(acc.at[pl.ds(r, CH)], buf)
      pltpu.sync_copy(buf, out.at[c, pl.ds(r, CH)])

  return agg


@functools.lru_cache(maxsize=None)
def _make_sc_deg():
  @functools.partial(
      pl.kernel,
      out_type=jax.ShapeDtypeStruct((NC, ROWS, 128), jnp.float32),
      mesh=plsc.VectorSubcoreMesh(core_axis_name="c", subcore_axis_name="s",
                                  num_cores=NC, num_subcores=NS),
      scratch_types=[
          pltpu.VMEM((CH,), jnp.int32),
          pltpu.VMEM((CH, 128), jnp.float32),
          pltpu.VMEM_SHARED((ROWS, 128), jnp.float32),
      ],
  )
  def deg(dsts, out, didx, buf, acc):
    c = lax.axis_index("c")
    s = lax.axis_index("s")
    z16 = jnp.zeros((16,), jnp.float32)
    o16 = jnp.ones((16,), jnp.float32)

    def zrow(i, _):
      for j in range(8):
        buf[i, pl.ds(j * 16, 16)] = z16
      return 0

    lax.fori_loop(0, CH, zrow, 0)
    for k in range(RPS // CH):
      pltpu.sync_copy(buf, acc.at[pl.ds(s * RPS + k * CH, CH)])
    plsc.subcore_barrier()

    def orow(i, _):
      for j in range(8):
        buf[i, pl.ds(j * 16, 16)] = o16
      return 0

    lax.fori_loop(0, CH, orow, 0)

    ebase = s * EPW

    def body(g, _):
      pltpu.sync_copy(dsts.at[c, pl.ds(ebase + g * CH, CH)], didx)
      pltpu.sync_copy(buf, acc.at[didx], add=True)
      return 0

    lax.fori_loop(0, EPW // CH, body, 0)
    plsc.subcore_barrier()

    for k in range(RPS // CH):
      r = s * RPS + k * CH
      pltpu.sync_copy(acc.at[pl.ds(r, CH)], buf)
      pltpu.sync_copy(buf, out.at[c, pl.ds(r, CH)])

  return deg


def _tc_pre_body(x_ref, w_ref, dsum_ref, dinv_ref, p_ref):
  dv = lax.rsqrt(dsum_ref[...])
  dinv_ref[...] = dv
  p = jnp.dot(x_ref[...], w_ref[...], preferred_element_type=jnp.float32) * dv
  p_ref[0] = p[:, : HID // 2]
  p_ref[1] = p[:, HID // 2 :]


def _tc_pre(x, W1, dsum):
  return pl.pallas_call(
      _tc_pre_body,
      grid=(GRID,),
      in_specs=[
          pl.BlockSpec((RB, IN), lambda i: (i, 0)),
          pl.BlockSpec((IN, HID), lambda i: (0, 0)),
          pl.BlockSpec((RB, 1), lambda i: (i, 0)),
      ],
      out_specs=[
          pl.BlockSpec((RB, 1), lambda i: (i, 0)),
          pl.BlockSpec((2, RB, HID // 2), lambda i: (0, i, 0)),
      ],
      out_shape=[
          jax.ShapeDtypeStruct((N, 1), jnp.float32),
          jax.ShapeDtypeStruct((2, N, HID // 2), jnp.float32),
      ],
  )(x, W1, dsum)


def _tc_layer_body(k, agg_ref, p_ref, dinv_ref, b_ref, w_ref, sw_ref,
                   fused_ref, pn_ref, fout_ref):
  dv = dinv_ref[...]
  h = jnp.concatenate([agg_ref[0] + p_ref[0], agg_ref[1] + p_ref[1]], axis=-1)
  h = jnp.maximum(h * dv + b_ref[...], 0.0)
  fout_ref[...] = fused_ref[...] + sw_ref[k] * h
  pn = jnp.dot(h, w_ref[...], preferred_element_type=jnp.float32) * dv
  pn_ref[0] = pn[:, : HID // 2]
  pn_ref[1] = pn[:, HID // 2 :]


def _tc_layer(k, agg, P, dinv, b, Wn, sw, fused):
  return pl.pallas_call(
      functools.partial(_tc_layer_body, k),
      grid=(GRID,),
      in_specs=[
          pl.BlockSpec((2, RB, HID // 2), lambda i: (0, i, 0)),
          pl.BlockSpec((2, RB, HID // 2), lambda i: (0, i, 0)),
          pl.BlockSpec((RB, 1), lambda i: (i, 0)),
          pl.BlockSpec((1, HID), lambda i: (0, 0)),
          pl.BlockSpec((HID, HID), lambda i: (0, 0)),
          pl.BlockSpec(memory_space=pltpu.SMEM),
          pl.BlockSpec((RB, HID), lambda i: (i, 0)),
      ],
      out_specs=[
          pl.BlockSpec((2, RB, HID // 2), lambda i: (0, i, 0)),
          pl.BlockSpec((RB, HID), lambda i: (i, 0)),
      ],
      out_shape=[
          jax.ShapeDtypeStruct((2, N, HID // 2), jnp.float32),
          jax.ShapeDtypeStruct((N, HID), jnp.float32),
      ],
  )(agg, P, dinv, b, Wn, sw, fused)


def _tc_layer4_body(agg_ref, p_ref, dinv_ref, b_ref, wf_ref, sw_ref,
                    fused_ref, pf_ref):
  dv = dinv_ref[...]
  h = jnp.concatenate([agg_ref[0] + p_ref[0], agg_ref[1] + p_ref[1]], axis=-1)
  h = jnp.maximum(h * dv + b_ref[...], 0.0)
  fused = fused_ref[...] + sw_ref[3] * h
  pf_ref[...] = jnp.dot(
      fused, wf_ref[...], preferred_element_type=jnp.float32) * dv


def _tc_layer4(agg, P, dinv, b, Wf, sw, fused):
  return pl.pallas_call(
      _tc_layer4_body,
      grid=(GRID,),
      in_specs=[
          pl.BlockSpec((2, RB, HID // 2), lambda i: (0, i, 0)),
          pl.BlockSpec((2, RB, HID // 2), lambda i: (0, i, 0)),
          pl.BlockSpec((RB, 1), lambda i: (i, 0)),
          pl.BlockSpec((1, HID), lambda i: (0, 0)),
          pl.BlockSpec((HID, OUT), lambda i: (0, 0)),
          pl.BlockSpec(memory_space=pltpu.SMEM),
          pl.BlockSpec((RB, HID), lambda i: (i, 0)),
      ],
      out_specs=[
          pl.BlockSpec((RB, OUT), lambda i: (i, 0)),
      ],
      out_shape=[
          jax.ShapeDtypeStruct((N, OUT), jnp.float32),
      ],
  )(agg, P, dinv, b, Wf, sw, fused)[0]


def _tc_final_body(agg_ref, p_ref, dinv_ref, b_ref, out_ref):
  o = agg_ref[0] + agg_ref[1] + p_ref[...]
  out_ref[...] = o * dinv_ref[...] + b_ref[...]


def _tc_final(agg, P, dinv, b):
  return pl.pallas_call(
      _tc_final_body,
      grid=(GRID,),
      in_specs=[
          pl.BlockSpec((2, RB, OUT), lambda i: (0, i, 0)),
          pl.BlockSpec((RB, OUT), lambda i: (i, 0)),
          pl.BlockSpec((RB, 1), lambda i: (i, 0)),
          pl.BlockSpec((1, OUT), lambda i: (0, 0)),
      ],
      out_specs=pl.BlockSpec((RB, OUT), lambda i: (i, 0)),
      out_shape=jax.ShapeDtypeStruct((N, OUT), jnp.float32),
  )(agg, P, dinv, b)


def kernel(x, edge_index, W1, b1, W2, b2, W3, b3, W4, b4, Wf, bf,
           scale_weights):
  src = edge_index[0]
  dst = edge_index[1]

  pad = EPS - E // NS
  srcp = jnp.concatenate(
      [src.reshape(NS, E // NS),
       jnp.zeros((NS, pad), jnp.int32)], axis=1).reshape(-1)
  dstp = jnp.concatenate(
      [dst.reshape(NS, E // NS),
       jnp.full((NS, pad), DUMMY, jnp.int32)], axis=1).reshape(-1)
  srcs2 = jnp.stack([srcp, srcp + N])

  padw = EPW - E // (NC * NS)
  dstw = jnp.concatenate(
      [dst.reshape(NC * NS, E // (NC * NS)),
       jnp.full((NC * NS, padw), DUMMY, jnp.int32)], axis=1)
  dstw = dstw.reshape(NC, NS * EPW)
  srcw = jnp.concatenate(
      [src.reshape(NC * NS, E // (NC * NS)),
       jnp.zeros((NC * NS, padw), jnp.int32)], axis=1)
  srcw = srcw.reshape(NC, NS * EPW)

  deg2 = _make_sc_deg()(dstw)
  dsum = (deg2[0, :N, 0] + deg2[1, :N, 0] + 1.0).reshape(N, 1)

  dinv, P = _tc_pre(x, W1, dsum)

  sw = jax.nn.softmax(scale_weights)
  fused = jnp.zeros((N, HID), jnp.float32)

  sc_agg128 = _make_sc_agg(HID // 2)
  for k, (b, Wn) in enumerate(((b1, W2), (b2, W3), (b3, W4))):
    agg = sc_agg128(P.reshape(2 * N, HID // 2), srcs2, dstp)
    P, fused = _tc_layer(k, agg, P, dinv, b.reshape(1, HID), Wn, sw, fused)

  agg = sc_agg128(P.reshape(2 * N, HID // 2), srcs2, dstp)
  Pf = _tc_layer4(agg, P, dinv, b4.reshape(1, HID), Wf, sw, fused)

  aggf = _make_sc_agg_full()(Pf, srcw, dstw)
  return _tc_final(aggf, Pf, dinv, bf.reshape(1, OUT))
